# baseline (speedup 1.0000x reference)
import functools

import jax
import jax.numpy as jnp
from jax import lax
from jax.experimental import pallas as pl
from jax.experimental.pallas import tpu as pltpu
from jax.experimental.pallas import tpu_sc as plsc


def _nn_body(p_ref, qt2_ref, q2_ref, p2_ref, out_ref,
             bv_ref, bi_ref, *, tn):
    j = pl.program_id(0)
    dots2 = jnp.dot(p_ref[...], qt2_ref[...],
                    preferred_element_type=jnp.float32)
    q2g = q2_ref[...]
    nk = tn // 8
    acc_v = (q2g + p2_ref[0:8, :]) - dots2[0:8, :]
    acc_i = jnp.zeros((8, q2g.shape[1]), jnp.float32)
    for k in range(1, nk):
        dk = (q2g + p2_ref[8 * k:8 * k + 8, :]) - dots2[8 * k:8 * k + 8, :]
        sel = dk < acc_v
        acc_v = jnp.minimum(acc_v, dk)
        acc_i = jnp.where(sel, jnp.float32(k), acc_i)
    m = jnp.min(acc_v, axis=0, keepdims=True)
    si = lax.broadcasted_iota(jnp.int32, acc_v.shape, 0).astype(jnp.float32)
    r = acc_i * 8.0 + si
    li = jnp.min(jnp.where(acc_v == m, r, jnp.float32(tn)),
                 axis=0, keepdims=True)
    gi = li + (j * tn).astype(jnp.float32)

    @pl.when(j == 0)
    def _():
        bv_ref[...] = m
        bi_ref[...] = gi

    @pl.when(j > 0)
    def _():
        better = m < bv_ref[...]
        bv_ref[...] = jnp.where(better, m, bv_ref[...])
        bi_ref[...] = jnp.where(better, gi, bi_ref[...])

    @pl.when(j == pl.num_programs(0) - 1)
    def _():
        out_ref[...] = bi_ref[...].astype(jnp.int32)


def _nn_indices(qt2, q2r, pos_pad, p2c, tn):
    qn = qt2.shape[1]
    n_tiles = pos_pad.shape[0] // tn
    out = pl.pallas_call(
        functools.partial(_nn_body, tn=tn),
        grid=(n_tiles,),
        in_specs=[
            pl.BlockSpec((tn, 3), lambda j: (j, 0)),
            pl.BlockSpec((3, qn), lambda j: (0, 0)),
            pl.BlockSpec((1, qn), lambda j: (0, 0)),
            pl.BlockSpec((tn, 1), lambda j: (j, 0)),
        ],
        out_specs=pl.BlockSpec((1, qn), lambda j: (0, 0)),
        out_shape=jax.ShapeDtypeStruct((1, qn), jnp.int32),
        scratch_shapes=[
            pltpu.VMEM((1, qn), jnp.float32),
            pltpu.VMEM((1, qn), jnp.float32),
        ],
    )(pos_pad, qt2, q2r, p2c)
    return out.reshape(qn)


def _sc_gather_rows(table, idx):
    info = plsc.get_sparse_core_info()
    nc, ns = info.num_cores, info.num_subcores
    nw = nc * ns
    qn = idx.shape[0]
    e = table.shape[1]
    b_per_w = qn // nw
    mesh = plsc.VectorSubcoreMesh(core_axis_name="c", subcore_axis_name="s")

    @functools.partial(
        pl.kernel,
        mesh=mesh,
        out_type=jax.ShapeDtypeStruct((qn, e), jnp.float32),
        scratch_types=[
            pltpu.VMEM((b_per_w,), jnp.int32),
            pltpu.VMEM((b_per_w, e), jnp.float32),
            pltpu.SemaphoreType.DMA,
        ],
    )
    def gather_k(table_hbm, idx_hbm, out_hbm, idx_v, rows_v, sem):
        wid = lax.axis_index("s") * nc + lax.axis_index("c")
        base = wid * b_per_w
        pltpu.sync_copy(idx_hbm.at[pl.ds(base, b_per_w)], idx_v)
        pltpu.async_copy(table_hbm.at[idx_v], rows_v, sem).wait()
        pltpu.sync_copy(rows_v, out_hbm.at[pl.ds(base, b_per_w)])

    return gather_k(table, idx)


def _expand_body(g_ref, t_ref, out_ref):
    g = g_ref[...]
    t = t_ref[...]
    out_ref[...] = g[:, None, :] + t[None, :, :]


def _expand_add(gathered, temporal, qb):
    qn, e = gathered.shape
    t = temporal.shape[0]
    return pl.pallas_call(
        _expand_body,
        grid=(qn // qb,),
        in_specs=[
            pl.BlockSpec((qb, e), lambda i: (i, 0)),
            pl.BlockSpec((t, e), lambda i: (0, 0)),
        ],
        out_specs=pl.BlockSpec((qb, t, e), lambda i: (i, 0, 0)),
        out_shape=jax.ShapeDtypeStruct((qn, t, e), jnp.float32),
    )(gathered, temporal)


def kernel(pos, positions, spatial_table, temporal_table):
    b, c, _ = pos.shape
    n, e = spatial_table.shape
    t = temporal_table.shape[0]
    qn = b * c

    q = pos.reshape(qn, 3)
    q2 = jnp.sum(pos * pos, axis=-1, keepdims=True).reshape(qn, 1)
    p2 = jnp.sum(positions * positions, axis=-1)

    tn = 1024
    n_tiles = -(-n // tn)
    n_pad = n_tiles * tn
    pos_pad = jnp.pad(positions, ((0, n_pad - n), (0, 0)))
    p2c = jnp.pad(p2, (0, n_pad - n), constant_values=1e30).reshape(n_pad, 1)
    qt2 = 2.0 * q.T
    q2r = q2.reshape(1, qn)

    idx = _nn_indices(qt2, q2r, pos_pad, p2c, tn)
    gathered = _sc_gather_rows(spatial_table, idx)
    out = _expand_add(gathered, temporal_table, 128)
    return out.reshape(b, c * t, e)


# speedup vs baseline: 1.4582x; 1.4582x over previous
import functools

import jax
import jax.numpy as jnp
from jax import lax
from jax.experimental import pallas as pl
from jax.experimental.pallas import tpu as pltpu
from jax.experimental.pallas import tpu_sc as plsc


def _nn_body(p_ref, qt2_ref, q2_ref, p2_ref, out_ref,
             bv_ref, bi_ref, *, tn):
    j = pl.program_id(0)
    dots2 = lax.dot_general(p_ref[...], qt2_ref[...],
                            (((0,), (0,)), ((), ())),
                            preferred_element_type=jnp.float32)
    q2g = q2_ref[...]
    p2v = p2_ref[0]
    nk = tn // 8
    acc_v = (q2g + p2v[:, 0:1]) - dots2[0:8, :]
    acc_i = jnp.zeros((8, q2g.shape[1]), jnp.float32)
    for k in range(1, nk):
        dk = (q2g + p2v[:, k:k + 1]) - dots2[8 * k:8 * k + 8, :]
        sel = dk < acc_v
        acc_v = jnp.minimum(acc_v, dk)
        acc_i = jnp.where(sel, jnp.float32(k), acc_i)
    m = jnp.min(acc_v, axis=0, keepdims=True)
    si = lax.broadcasted_iota(jnp.int32, acc_v.shape, 0).astype(jnp.float32)
    r = acc_i * 8.0 + si
    li = jnp.min(jnp.where(acc_v == m, r, jnp.float32(tn)),
                 axis=0, keepdims=True)
    gi = li + (j * tn).astype(jnp.float32)

    @pl.when(j == 0)
    def _():
        bv_ref[...] = m
        bi_ref[...] = gi

    @pl.when(j > 0)
    def _():
        better = m < bv_ref[...]
        bv_ref[...] = jnp.where(better, m, bv_ref[...])
        bi_ref[...] = jnp.where(better, gi, bi_ref[...])

    @pl.when(j == pl.num_programs(0) - 1)
    def _():
        out_ref[...] = bi_ref[...].astype(jnp.int32)


def _nn_indices(qt2, q2r, pos3, p2pack, tn):
    qn = qt2.shape[1]
    n_tiles = pos3.shape[1] // tn
    out = pl.pallas_call(
        functools.partial(_nn_body, tn=tn),
        grid=(n_tiles,),
        in_specs=[
            pl.BlockSpec((3, tn), lambda j: (0, j)),
            pl.BlockSpec((3, qn), lambda j: (0, 0)),
            pl.BlockSpec((1, qn), lambda j: (0, 0)),
            pl.BlockSpec((1, 8, tn // 8), lambda j: (j, 0, 0)),
        ],
        out_specs=pl.BlockSpec((1, qn), lambda j: (0, 0)),
        out_shape=jax.ShapeDtypeStruct((1, qn), jnp.int32),
        scratch_shapes=[
            pltpu.VMEM((1, qn), jnp.float32),
            pltpu.VMEM((1, qn), jnp.float32),
        ],
    )(pos3, qt2, q2r, p2pack)
    return out.reshape(qn)


def _sc_gather_rows(table, idx):
    info = plsc.get_sparse_core_info()
    nc, ns = info.num_cores, info.num_subcores
    nw = nc * ns
    qn = idx.shape[0]
    e = table.shape[1]
    b_per_w = qn // nw
    mesh = plsc.VectorSubcoreMesh(core_axis_name="c", subcore_axis_name="s")

    @functools.partial(
        pl.kernel,
        mesh=mesh,
        out_type=jax.ShapeDtypeStruct((qn, e), jnp.float32),
        scratch_types=[
            pltpu.VMEM((b_per_w,), jnp.int32),
            pltpu.VMEM((b_per_w, e), jnp.float32),
            pltpu.SemaphoreType.DMA,
        ],
    )
    def gather_k(table_hbm, idx_hbm, out_hbm, idx_v, rows_v, sem):
        wid = lax.axis_index("s") * nc + lax.axis_index("c")
        base = wid * b_per_w
        pltpu.sync_copy(idx_hbm.at[pl.ds(base, b_per_w)], idx_v)
        pltpu.async_copy(table_hbm.at[idx_v], rows_v, sem).wait()
        pltpu.sync_copy(rows_v, out_hbm.at[pl.ds(base, b_per_w)])

    return gather_k(table, idx)


def _expand_body(g_ref, t_ref, out_ref):
    g = g_ref[...]
    t = t_ref[...]
    out_ref[...] = g[:, None, :] + t[None, :, :]


def _expand_add(gathered, temporal, qb):
    qn, e = gathered.shape
    t = temporal.shape[0]
    return pl.pallas_call(
        _expand_body,
        grid=(qn // qb,),
        in_specs=[
            pl.BlockSpec((qb, e), lambda i: (i, 0)),
            pl.BlockSpec((t, e), lambda i: (0, 0)),
        ],
        out_specs=pl.BlockSpec((qb, t, e), lambda i: (i, 0, 0)),
        out_shape=jax.ShapeDtypeStruct((qn, t, e), jnp.float32),
    )(gathered, temporal)


def kernel(pos, positions, spatial_table, temporal_table):
    b, c, _ = pos.shape
    n, e = spatial_table.shape
    t = temporal_table.shape[0]
    qn = b * c

    q = pos.reshape(qn, 3)
    q2 = jnp.sum(pos * pos, axis=-1, keepdims=True).reshape(qn, 1)
    p2 = jnp.sum(positions * positions, axis=-1)

    tn = 1024
    n_tiles = -(-n // tn)
    n_pad = n_tiles * tn
    pos3 = jnp.pad(positions, ((0, n_pad - n), (0, 0))).T
    p2pack = (jnp.pad(p2, (0, n_pad - n), constant_values=1e30)
              .reshape(n_tiles, tn // 8, 8).transpose(0, 2, 1))
    qt2 = 2.0 * q.T
    q2r = q2.reshape(1, qn)

    idx = _nn_indices(qt2, q2r, pos3, p2pack, tn)
    gathered = _sc_gather_rows(spatial_table, idx)
    out = _expand_add(gathered, temporal_table, 128)
    return out.reshape(b, c * t, e)


# speedup vs baseline: 1.5264x; 1.0468x over previous
import functools

import jax
import jax.numpy as jnp
from jax import lax
from jax.experimental import pallas as pl
from jax.experimental.pallas import tpu as pltpu
from jax.experimental.pallas import tpu_sc as plsc


def _nn_body(p_ref, qt2_ref, q2_ref, p2_ref, out_ref,
             bv_ref, bi_ref, *, tn):
    j = pl.program_id(0)
    dots2 = lax.dot_general(p_ref[...], qt2_ref[...],
                            (((0,), (0,)), ((), ())),
                            preferred_element_type=jnp.float32)
    q2g = q2_ref[...]
    p2v = p2_ref[0]
    nk = tn // 8
    acc_v = (q2g + p2v[:, 0:1]) - dots2[0:8, :]
    acc_i = jnp.zeros((8, q2g.shape[1]), jnp.float32)
    for k in range(1, nk):
        dk = (q2g + p2v[:, k:k + 1]) - dots2[8 * k:8 * k + 8, :]
        sel = dk < acc_v
        acc_v = jnp.minimum(acc_v, dk)
        acc_i = jnp.where(sel, jnp.float32(k), acc_i)
    m = jnp.min(acc_v, axis=0, keepdims=True)
    si = lax.broadcasted_iota(jnp.int32, acc_v.shape, 0).astype(jnp.float32)
    r = acc_i * 8.0 + si
    li = jnp.min(jnp.where(acc_v == m, r, jnp.float32(tn)),
                 axis=0, keepdims=True)
    gi = li + (j * tn).astype(jnp.float32)

    @pl.when(j == 0)
    def _():
        bv_ref[...] = m
        bi_ref[...] = gi

    @pl.when(j > 0)
    def _():
        better = m < bv_ref[...]
        bv_ref[...] = jnp.where(better, m, bv_ref[...])
        bi_ref[...] = jnp.where(better, gi, bi_ref[...])

    @pl.when(j == pl.num_programs(0) - 1)
    def _():
        out_ref[...] = bi_ref[...].astype(jnp.int32)


def _nn_indices(qt2, q2r, pos3, p2pack, tn):
    qn = qt2.shape[1]
    n_tiles = pos3.shape[1] // tn
    out = pl.pallas_call(
        functools.partial(_nn_body, tn=tn),
        grid=(n_tiles,),
        in_specs=[
            pl.BlockSpec((3, tn), lambda j: (0, j)),
            pl.BlockSpec((3, qn), lambda j: (0, 0)),
            pl.BlockSpec((1, qn), lambda j: (0, 0)),
            pl.BlockSpec((1, 8, tn // 8), lambda j: (j, 0, 0)),
        ],
        out_specs=pl.BlockSpec((1, qn), lambda j: (0, 0)),
        out_shape=jax.ShapeDtypeStruct((1, qn), jnp.int32),
        scratch_shapes=[
            pltpu.VMEM((1, qn), jnp.float32),
            pltpu.VMEM((1, qn), jnp.float32),
        ],
    )(pos3, qt2, q2r, p2pack)
    return out.reshape(qn)


def _sc_gather_rows(table, idx):
    info = plsc.get_sparse_core_info()
    nc, ns = info.num_cores, info.num_subcores
    nw = nc * ns
    qn = idx.shape[0]
    e = table.shape[1]
    b_per_w = qn // nw
    mesh = plsc.VectorSubcoreMesh(core_axis_name="c", subcore_axis_name="s")

    @functools.partial(
        pl.kernel,
        mesh=mesh,
        out_type=jax.ShapeDtypeStruct((qn, e), jnp.float32),
        scratch_types=[
            pltpu.VMEM((b_per_w,), jnp.int32),
            pltpu.VMEM((b_per_w, e), jnp.float32),
            pltpu.SemaphoreType.DMA,
        ],
    )
    def gather_k(table_hbm, idx_hbm, out_hbm, idx_v, rows_v, sem):
        wid = lax.axis_index("s") * nc + lax.axis_index("c")
        base = wid * b_per_w
        pltpu.sync_copy(idx_hbm.at[pl.ds(base, b_per_w)], idx_v)
        pltpu.async_copy(table_hbm.at[idx_v], rows_v, sem).wait()
        pltpu.sync_copy(rows_v, out_hbm.at[pl.ds(base, b_per_w)])

    return gather_k(table, idx)


def _expand_body(g_ref, t_ref, out_ref):
    g = g_ref[...]
    t = t_ref[...]
    out_ref[...] = g[:, None, :] + t[None, :, :]


def _expand_add(gathered, temporal, qb):
    qn, e = gathered.shape
    t = temporal.shape[0]
    return pl.pallas_call(
        _expand_body,
        grid=(qn // qb,),
        in_specs=[
            pl.BlockSpec((qb, e), lambda i: (i, 0)),
            pl.BlockSpec((t, e), lambda i: (0, 0)),
        ],
        out_specs=pl.BlockSpec((qb, t, e), lambda i: (i, 0, 0)),
        out_shape=jax.ShapeDtypeStruct((qn, t, e), jnp.float32),
    )(gathered, temporal)


def kernel(pos, positions, spatial_table, temporal_table):
    b, c, _ = pos.shape
    n, e = spatial_table.shape
    t = temporal_table.shape[0]
    qn = b * c

    q = pos.reshape(qn, 3)
    q2 = jnp.sum(pos * pos, axis=-1, keepdims=True).reshape(qn, 1)
    p2 = jnp.sum(positions * positions, axis=-1)

    tn = 2048
    n_tiles = -(-n // tn)
    n_pad = n_tiles * tn
    pos3 = jnp.pad(positions, ((0, n_pad - n), (0, 0))).T
    p2pack = (jnp.pad(p2, (0, n_pad - n), constant_values=1e30)
              .reshape(n_tiles, tn // 8, 8).transpose(0, 2, 1))
    qt2 = 2.0 * q.T
    q2r = q2.reshape(1, qn)

    idx = _nn_indices(qt2, q2r, pos3, p2pack, tn)
    gathered = _sc_gather_rows(spatial_table, idx)
    out = _expand_add(gathered, temporal_table, 128)
    return out.reshape(b, c * t, e)


# speedup vs baseline: 1.5275x; 1.0007x over previous
import functools

import jax
import jax.numpy as jnp
from jax import lax
from jax.experimental import pallas as pl
from jax.experimental.pallas import tpu as pltpu
from jax.experimental.pallas import tpu_sc as plsc


def _nn_body(p_ref, qt2_ref, q2_ref, p2_ref, out_ref,
             bv_ref, bi_ref, *, tn):
    j = pl.program_id(0)
    dots2 = lax.dot_general(p_ref[...], qt2_ref[...],
                            (((0,), (0,)), ((), ())),
                            preferred_element_type=jnp.float32)
    q2g = q2_ref[...]
    p2v = p2_ref[0]
    nk = tn // 8
    acc_v = (q2g + p2v[:, 0:1]) - dots2[0:8, :]
    acc_i = jnp.zeros((8, q2g.shape[1]), jnp.float32)
    for k in range(1, nk):
        dk = (q2g + p2v[:, k:k + 1]) - dots2[8 * k:8 * k + 8, :]
        sel = dk < acc_v
        acc_v = jnp.minimum(acc_v, dk)
        acc_i = jnp.where(sel, jnp.float32(k), acc_i)
    m = jnp.min(acc_v, axis=0, keepdims=True)
    si = lax.broadcasted_iota(jnp.int32, acc_v.shape, 0).astype(jnp.float32)
    r = acc_i * 8.0 + si
    li = jnp.min(jnp.where(acc_v == m, r, jnp.float32(tn)),
                 axis=0, keepdims=True)
    gi = li + (j * tn).astype(jnp.float32)

    @pl.when(j == 0)
    def _():
        bv_ref[...] = m
        bi_ref[...] = gi

    @pl.when(j > 0)
    def _():
        better = m < bv_ref[...]
        bv_ref[...] = jnp.where(better, m, bv_ref[...])
        bi_ref[...] = jnp.where(better, gi, bi_ref[...])

    @pl.when(j == pl.num_programs(0) - 1)
    def _():
        out_ref[...] = bi_ref[...].astype(jnp.int32)


def _nn_indices(qt2, q2r, pos3, p2pack, tn):
    qn = qt2.shape[1]
    n_tiles = pos3.shape[1] // tn
    out = pl.pallas_call(
        functools.partial(_nn_body, tn=tn),
        grid=(n_tiles,),
        in_specs=[
            pl.BlockSpec((3, tn), lambda j: (0, j)),
            pl.BlockSpec((3, qn), lambda j: (0, 0)),
            pl.BlockSpec((1, qn), lambda j: (0, 0)),
            pl.BlockSpec((1, 8, tn // 8), lambda j: (j, 0, 0)),
        ],
        out_specs=pl.BlockSpec((1, qn), lambda j: (0, 0)),
        out_shape=jax.ShapeDtypeStruct((1, qn), jnp.int32),
        scratch_shapes=[
            pltpu.VMEM((1, qn), jnp.float32),
            pltpu.VMEM((1, qn), jnp.float32),
        ],
    )(pos3, qt2, q2r, p2pack)
    return out.reshape(qn)


def _sc_gather_rows(table, idx):
    info = plsc.get_sparse_core_info()
    nc, ns = info.num_cores, info.num_subcores
    nw = nc * ns
    qn = idx.shape[0]
    e = table.shape[1]
    b_per_w = qn // nw
    mesh = plsc.VectorSubcoreMesh(core_axis_name="c", subcore_axis_name="s")

    @functools.partial(
        pl.kernel,
        mesh=mesh,
        out_type=jax.ShapeDtypeStruct((qn, e), jnp.float32),
        scratch_types=[
            pltpu.VMEM((b_per_w,), jnp.int32),
            pltpu.VMEM((b_per_w, e), jnp.float32),
            pltpu.SemaphoreType.DMA,
        ],
    )
    def gather_k(table_hbm, idx_hbm, out_hbm, idx_v, rows_v, sem):
        wid = lax.axis_index("s") * nc + lax.axis_index("c")
        base = wid * b_per_w
        pltpu.sync_copy(idx_hbm.at[pl.ds(base, b_per_w)], idx_v)
        pltpu.async_copy(table_hbm.at[idx_v], rows_v, sem).wait()
        pltpu.sync_copy(rows_v, out_hbm.at[pl.ds(base, b_per_w)])

    return gather_k(table, idx)


def _expand_body(g_ref, t_ref, out_ref, *, qb):
    g = g_ref[...]
    t = t_ref[...]
    r = g[:, None, :] + t[None, :, :]
    out_ref[...] = r.reshape(1, qb * t_ref.shape[0], r.shape[-1])


def _expand_add(gathered, temporal, qb, b, c):
    qn, e = gathered.shape
    t = temporal.shape[0]
    blocks_per_b = c // qb
    return pl.pallas_call(
        functools.partial(_expand_body, qb=qb),
        grid=(qn // qb,),
        in_specs=[
            pl.BlockSpec((qb, e), lambda i: (i, 0)),
            pl.BlockSpec((t, e), lambda i: (0, 0)),
        ],
        out_specs=pl.BlockSpec(
            (1, qb * t, e),
            lambda i: (i // blocks_per_b, i % blocks_per_b, 0)),
        out_shape=jax.ShapeDtypeStruct((b, c * t, e), jnp.float32),
    )(gathered, temporal)


def kernel(pos, positions, spatial_table, temporal_table):
    b, c, _ = pos.shape
    n, e = spatial_table.shape
    t = temporal_table.shape[0]
    qn = b * c

    q = pos.reshape(qn, 3)
    q2 = jnp.sum(pos * pos, axis=-1, keepdims=True).reshape(qn, 1)
    p2 = jnp.sum(positions * positions, axis=-1)

    tn = 2048
    n_tiles = -(-n // tn)
    n_pad = n_tiles * tn
    pos3 = jnp.pad(positions, ((0, n_pad - n), (0, 0))).T
    p2pack = (jnp.pad(p2, (0, n_pad - n), constant_values=1e30)
              .reshape(n_tiles, tn // 8, 8).transpose(0, 2, 1))
    qt2 = 2.0 * q.T
    q2r = q2.reshape(1, qn)

    idx = _nn_indices(qt2, q2r, pos3, p2pack, tn)
    gathered = _sc_gather_rows(spatial_table, idx)
    return _expand_add(gathered, temporal_table, 128, b, c)
